# baseline (speedup 1.0000x reference)
import jax
import jax.numpy as jnp
from jax.experimental import pallas as pl


def _mlp(ps, x):
    for i, (W, b) in enumerate(ps):
        x = x @ W + b
        if i < len(ps) - 1:
            x = jax.nn.silu(x)
    return x


def _dec_kernel(x_ref, w1, b1, w2, b2, w3, b3, o_ref):
    h = jax.nn.silu(x_ref[...] @ w1[...] + b1[...])
    h = jax.nn.silu(h @ w2[...] + b2[...])
    o_ref[...] = h @ w3[...] + b3[...]


def kernel(z, n, edge_index, q_0, params):
    src = edge_index[0]
    dest = edge_index[1]
    x = jnp.concatenate([z, n], axis=1)
    u = q_0[src] - q_0[dest]
    u_norm = jnp.linalg.norm(u, axis=1, keepdims=True)
    edge_attr = jnp.concatenate([u, u_norm], axis=1)
    x = _mlp(params['enc_node'], x)
    edge_attr = _mlp(params['enc_edge'], edge_attr)
    for p in range(2):
        ea_new = _mlp(params['edge_%d' % p],
                      jnp.concatenate([edge_attr, x[src], x[dest]], axis=1))
        agg = jax.ops.segment_sum(ea_new, dest, num_segments=x.shape[0])
        x_new = _mlp(params['node_%d' % p], jnp.concatenate([x, agg], axis=1))
        x = x + x_new
        edge_attr = edge_attr + ea_new
    (w1, b1), (w2, b2), (w3, b3) = params['dec']
    N = x.shape[0]
    B = 20000
    out = pl.pallas_call(
        _dec_kernel,
        grid=(N // B,),
        in_specs=[pl.BlockSpec((B, 32), lambda i: (i, 0))] + [
            pl.BlockSpec(w.shape, lambda i, _r=len(w.shape): (0,) * _r)
            for w in (w1, b1, w2, b2, w3, b3)
        ],
        out_specs=pl.BlockSpec((B, 3), lambda i: (i, 0)),
        out_shape=jax.ShapeDtypeStruct((N, 3), jnp.float32),
    )(x, w1, b1, w2, b2, w3, b3)
    return out


# speedup vs baseline: 3.8626x; 3.8626x over previous
import functools

import jax
import jax.numpy as jnp
from jax import lax
from jax.experimental import pallas as pl
from jax.experimental.pallas import tpu as pltpu
from jax.experimental.pallas import tpu_sc as plsc

H = 32
_SILU = jax.nn.silu

_NC = 2
_NS = 16
_CW = 128
_KR = 10


def _node_enc_body(z, n, wz, wn, b1, w2, b2, w3, b3, ws, wd,
                   x0_o, s_o, d_o):
    h = _SILU(z[...] @ wz[...] + n[...] @ wn[...] + b1[...])
    h = _SILU(h @ w2[...] + b2[...])
    x0 = h @ w3[...] + b3[...]
    x0_o[...] = x0
    s_o[...] = x0 @ ws[...]
    d_o[...] = x0 @ wd[...]


def _edge0_body(g, up, we1p, we1n, be1, we2, be2, we3, be3,
                w1e, b1, w2, b2, w3, b3, ea_o, eat_o):
    u = up[...]
    norm = jnp.sqrt(jnp.sum(u * u, axis=1, keepdims=True))
    e = _SILU(u @ we1p[...] + norm * we1n[...] + be1[...])
    e = _SILU(e @ we2[...] + be2[...])
    e = e @ we3[...] + be3[...]
    h = _SILU(e @ w1e[...] + g[...] + b1[...])
    h = _SILU(h @ w2[...] + b2[...])
    ea = h @ w3[...] + b3[...]
    ea_o[...] = ea
    eat_o[...] = e + ea


def _edge1_body(eat, g, w1e, b1, w2, b2, w3, b3, ea_o):
    h = _SILU(eat[...] @ w1e[...] + g[...] + b1[...])
    h = _SILU(h @ w2[...] + b2[...])
    ea_o[...] = h @ w3[...] + b3[...]


def _node_upd_body(x, a, w1x, w1a, b1, w2, b2, w3, b3, ws, wd,
                   x1_o, s_o, d_o):
    h = _SILU(x[...] @ w1x[...] + a[...] @ w1a[...] + b1[...])
    h = _SILU(h @ w2[...] + b2[...])
    x1 = x[...] + (h @ w3[...] + b3[...])
    x1_o[...] = x1
    s_o[...] = x1 @ ws[...]
    d_o[...] = x1 @ wd[...]


def _node_dec_body(x, a, w1x, w1a, b1, w2, b2, w3, b3,
                   wd1, bd1, wd2, bd2, wd3, bd3, o):
    h = _SILU(x[...] @ w1x[...] + a[...] @ w1a[...] + b1[...])
    h = _SILU(h @ w2[...] + b2[...])
    x2 = x[...] + (h @ w3[...] + b3[...])
    h = _SILU(x2 @ wd1[...] + bd1[...])
    h = _SILU(h @ wd2[...] + bd2[...])
    o[...] = h @ wd3[...] + bd3[...]


def _tc_call(body, n_rows, blk, data, weights, out_widths):
    specs = [pl.BlockSpec((blk, d.shape[1]), lambda i: (i, 0)) for d in data]
    specs += [pl.BlockSpec(w.shape, lambda i, _r=w.ndim: (0,) * _r)
              for w in weights]
    return pl.pallas_call(
        body,
        grid=(n_rows // blk,),
        in_specs=specs,
        out_specs=[pl.BlockSpec((blk, wd), lambda i: (i, 0))
                   for wd in out_widths],
        out_shape=[jax.ShapeDtypeStruct((n_rows, wd), jnp.float32)
                   for wd in out_widths],
    )(*data, *weights)


def _sc_gather(pairs, src2, dest2, n_chunks):
    mesh = plsc.VectorSubcoreMesh(core_axis_name="c", subcore_axis_name="s")
    ecnt = n_chunks * _KR * _CW
    dims = [ta.shape[1] for ta, _ in pairs]
    npair = len(dims)

    @functools.partial(
        pl.kernel, mesh=mesh,
        out_type=[jax.ShapeDtypeStruct((ecnt, d), jnp.float32) for d in dims],
        compiler_params=pltpu.CompilerParams(use_tc_tiling_on_sc=False),
        scratch_types=[
            pltpu.VMEM((_KR, _CW), jnp.int32),
            pltpu.VMEM((_KR, _CW), jnp.int32),
        ] + [pltpu.VMEM((_KR * _CW, d), jnp.float32) for d in dims] + [
            pltpu.SemaphoreType.DMA,
        ],
    )
    def k(*refs):
        tabs = refs[:2 * npair]
        src_h, dest_h = refs[2 * npair:2 * npair + 2]
        outs = refs[2 * npair + 2:2 * npair + 2 + npair]
        ia, ib = refs[2 * npair + 2 + npair:2 * npair + 4 + npair]
        rows_l = refs[2 * npair + 4 + npair:2 * npair + 4 + 2 * npair]
        sem = refs[-1]
        w = lax.axis_index("s") * _NC + lax.axis_index("c")

        @pl.loop(w, n_chunks, step=_NC * _NS)
        def _chunk(ch):
            pltpu.sync_copy(src_h.at[ch], ia)
            pltpu.sync_copy(dest_h.at[ch], ib)
            for t in range(npair):
                ta_h, tb_h, rows = tabs[2 * t], tabs[2 * t + 1], rows_l[t]
                cps = [pltpu.async_copy(ta_h.at[ia.at[j]],
                                        rows.at[pl.ds(j * _CW, _CW)], sem)
                       for j in range(_KR)]
                for cp in cps:
                    cp.wait()
                cps = [pltpu.async_copy(tb_h.at[ib.at[j]],
                                        rows.at[pl.ds(j * _CW, _CW)], sem,
                                        add=True)
                       for j in range(_KR)]
                for cp in cps:
                    cp.wait()
                pltpu.sync_copy(
                    rows, outs[t].at[pl.ds(ch * _KR * _CW, _KR * _CW)])

    flat = []
    for ta, tb in pairs:
        flat += [ta, tb]
    res = k(*flat, src2, dest2)
    return res if isinstance(res, (list, tuple)) else (res,)


def _sc_scatter(ea, dest2, zeros_h, n_nodes):
    mesh = plsc.VectorSubcoreMesh(core_axis_name="c", subcore_axis_name="s")
    kr = 4
    n_chunks = dest2.shape[0]
    rng = n_nodes // _NC
    sr = rng + 48
    zr = sr // _NS
    wb = (rng // _NS) // 8 * 8
    tail = rng - wb * _NS
    trash = rng

    @functools.partial(
        pl.kernel, mesh=mesh,
        out_type=jax.ShapeDtypeStruct((n_nodes, H), jnp.float32),
        compiler_params=pltpu.CompilerParams(use_tc_tiling_on_sc=False),
        scratch_types=[
            pltpu.VMEM_SHARED((sr, H), jnp.float32),
            pltpu.VMEM((kr, _CW), jnp.int32),
            pltpu.VMEM((kr, _CW), jnp.int32),
            pltpu.VMEM((kr * _CW, H), jnp.float32),
            pltpu.SemaphoreType.DMA,
        ],
    )
    def k(ea_h, dest_h, z_h, agg_h, acc, ib, i2, rows, sem):
        c = lax.axis_index("c")
        s = lax.axis_index("s")
        base = c * rng
        pltpu.sync_copy(z_h, acc.at[pl.ds(s * zr, zr)])
        plsc.subcore_barrier()

        @pl.loop(s, n_chunks, step=_NS)
        def _chunk(ch):
            pltpu.sync_copy(dest_h.at[ch], ib)
            pltpu.sync_copy(ea_h.at[pl.ds(ch * kr * _CW, kr * _CW)], rows)
            for j in range(kr):
                for q in range(_CW // 16):
                    dv = ib[j, pl.ds(q * 16, 16)]
                    rel = dv - base
                    ok = (rel >= 0) & (rel < rng)
                    i2[j, pl.ds(q * 16, 16)] = jnp.where(ok, rel, trash)
            for j in range(kr):
                pltpu.sync_copy(rows.at[pl.ds(j * _CW, _CW)],
                                acc.at[i2.at[j]], add=True)
        plsc.subcore_barrier()
        pltpu.sync_copy(acc.at[pl.ds(s * wb, wb)],
                        agg_h.at[pl.ds(base + s * wb, wb)])

        @pl.when(s == 0)
        def _tail():
            pltpu.sync_copy(acc.at[pl.ds(wb * _NS, tail)],
                            agg_h.at[pl.ds(base + wb * _NS, tail)])

    return k(ea, dest2, zeros_h)


def kernel(z, n, edge_index, q_0, params):
    N = z.shape[0]
    E = edge_index.shape[1]
    n_chunks = E // (_KR * _CW)
    src2 = edge_index[0].reshape(n_chunks, _KR, _CW)
    dest2 = edge_index[1].reshape(n_chunks, _KR, _CW)
    dest2s = edge_index[1].reshape(E // (4 * _CW), 4, _CW)
    rng = N // _NC
    zeros_h = jnp.zeros(((rng + 48) // _NS, H), jnp.float32)

    def pad16(a):
        return jnp.pad(a, ((0, 0), (0, 16 - a.shape[1])))

    z16, n16 = pad16(z), pad16(n)
    q16 = pad16(q_0)
    nq16 = pad16(-q_0)

    p = params
    (wz6, b1), (w2, b2), (w3, b3) = p['enc_node']
    (we1, be1), (we2, be2), (we3, be3) = p['enc_edge']
    wz16 = jnp.zeros((16, H), jnp.float32).at[:3].set(wz6[:3])
    wn16 = jnp.zeros((16, H), jnp.float32).at[:3].set(wz6[3:])
    we1p = jnp.zeros((16, H), jnp.float32).at[:3].set(we1[:3])

    def row(b):
        return b.reshape(1, -1)

    e0 = p['edge_0']
    e1 = p['edge_1']
    n0 = p['node_0']
    n1 = p['node_1']
    dec = p['dec']

    Bn = 4000
    Be = 6400

    x0, s0, d0 = _tc_call(
        _node_enc_body, N, Bn, [z16, n16],
        [wz16, wn16, row(b1), w2, row(b2), w3, row(b3),
         e0[0][0][H:2 * H], e0[0][0][2 * H:]],
        [H, H, H])

    g0, u0 = _sc_gather([(s0, d0), (q16, nq16)], src2, dest2, n_chunks)
    ea0, eat1 = _tc_call(
        _edge0_body, E, Be, [g0, u0],
        [we1p, row(we1[3]), row(be1), we2, row(be2), we3, row(be3),
         e0[0][0][:H], row(e0[0][1]), e0[1][0], row(e0[1][1]),
         e0[2][0], row(e0[2][1])],
        [H, H])
    agg0 = _sc_scatter(ea0, dest2s, zeros_h, N)
    x1, s1, d1 = _tc_call(
        _node_upd_body, N, Bn, [x0, agg0],
        [n0[0][0][:H], n0[0][0][H:], row(n0[0][1]), n0[1][0], row(n0[1][1]),
         n0[2][0], row(n0[2][1]), e1[0][0][H:2 * H], e1[0][0][2 * H:]],
        [H, H, H])

    (g1,) = _sc_gather([(s1, d1)], src2, dest2, n_chunks)
    (ea1,) = _tc_call(
        _edge1_body, E, Be, [eat1, g1],
        [e1[0][0][:H], row(e1[0][1]), e1[1][0], row(e1[1][1]),
         e1[2][0], row(e1[2][1])],
        [H])
    agg1 = _sc_scatter(ea1, dest2s, zeros_h, N)
    (out,) = _tc_call(
        _node_dec_body, N, Bn, [x1, agg1],
        [n1[0][0][:H], n1[0][0][H:], row(n1[0][1]), n1[1][0], row(n1[1][1]),
         n1[2][0], row(n1[2][1]),
         dec[0][0], row(dec[0][1]), dec[1][0], row(dec[1][1]),
         dec[2][0], row(dec[2][1])],
        [3])
    return out


# speedup vs baseline: 7.3227x; 1.8958x over previous
import functools

import jax
import jax.numpy as jnp
from jax import lax
from jax.experimental import pallas as pl
from jax.experimental.pallas import tpu as pltpu
from jax.experimental.pallas import tpu_sc as plsc

H = 32
_SILU = jax.nn.silu

_NC = 2
_NS = 16
_CW = 128
_KR = 10


def _node_enc_body(z, n, wz, wn, b1, w2, b2, w3, b3, ws, wd,
                   x0_o, s_o, d_o):
    h = _SILU(z[...] @ wz[...] + n[...] @ wn[...] + b1[...])
    h = _SILU(h @ w2[...] + b2[...])
    x0 = h @ w3[...] + b3[...]
    x0_o[...] = x0
    s_o[...] = x0 @ ws[...]
    d_o[...] = x0 @ wd[...]


def _edge0_body(g, up, msk, we1p, we1n, be1, we2, be2, we3, be3,
                w1e, b1, w2, b2, w3, b3, ea_o, eat_o):
    u = up[...]
    norm = jnp.sqrt((u * u) @ msk[...])
    e = _SILU(u @ we1p[...] + norm * we1n[...] + be1[...])
    e = _SILU(e @ we2[...] + be2[...])
    e = e @ we3[...] + be3[...]
    h = _SILU(e @ w1e[...] + g[...] + b1[...])
    h = _SILU(h @ w2[...] + b2[...])
    ea = h @ w3[...] + b3[...]
    ea_o[...] = ea
    eat_o[...] = e + ea


def _edge1_body(eat, g, w1e, b1, w2, b2, w3, b3, ea_o):
    h = _SILU(eat[...] @ w1e[...] + g[...] + b1[...])
    h = _SILU(h @ w2[...] + b2[...])
    ea_o[...] = h @ w3[...] + b3[...]


def _node_upd_body(x, a, w1x, w1a, b1, w2, b2, w3, b3, ws, wd,
                   x1_o, s_o, d_o):
    h = _SILU(x[...] @ w1x[...] + a[...] @ w1a[...] + b1[...])
    h = _SILU(h @ w2[...] + b2[...])
    x1 = x[...] + (h @ w3[...] + b3[...])
    x1_o[...] = x1
    s_o[...] = x1 @ ws[...]
    d_o[...] = x1 @ wd[...]


def _node_dec_body(x, a, w1x, w1a, b1, w2, b2, w3, b3,
                   wd1, bd1, wd2, bd2, wd3, bd3, o):
    h = _SILU(x[...] @ w1x[...] + a[...] @ w1a[...] + b1[...])
    h = _SILU(h @ w2[...] + b2[...])
    x2 = x[...] + (h @ w3[...] + b3[...])
    h = _SILU(x2 @ wd1[...] + bd1[...])
    h = _SILU(h @ wd2[...] + bd2[...])
    o[...] = h @ wd3[...] + bd3[...]


def _tc_call(body, n_rows, blk, data, weights, out_widths):
    specs = [pl.BlockSpec((blk, d.shape[1]), lambda i: (i, 0)) for d in data]
    specs += [pl.BlockSpec(w.shape, lambda i, _r=w.ndim: (0,) * _r)
              for w in weights]
    return pl.pallas_call(
        body,
        grid=(n_rows // blk,),
        in_specs=specs,
        out_specs=[pl.BlockSpec((blk, wd), lambda i: (i, 0))
                   for wd in out_widths],
        out_shape=[jax.ShapeDtypeStruct((n_rows, wd), jnp.float32)
                   for wd in out_widths],
    )(*data, *weights)


def _sc_gather(pairs, src2, dest2, n_chunks):
    mesh = plsc.VectorSubcoreMesh(core_axis_name="c", subcore_axis_name="s")
    ecnt = n_chunks * _KR * _CW
    dims = [ta.shape[1] for ta, _ in pairs]
    npair = len(dims)

    @functools.partial(
        pl.kernel, mesh=mesh,
        out_type=[jax.ShapeDtypeStruct((ecnt, d), jnp.float32) for d in dims],
        compiler_params=pltpu.CompilerParams(use_tc_tiling_on_sc=False),
        scratch_types=[
            pltpu.VMEM((_KR, _CW), jnp.int32),
            pltpu.VMEM((_KR, _CW), jnp.int32),
        ] + [pltpu.VMEM((_KR * _CW, d), jnp.float32) for d in dims] + [
            pltpu.SemaphoreType.DMA,
        ],
    )
    def k(*refs):
        tabs = refs[:2 * npair]
        src_h, dest_h = refs[2 * npair:2 * npair + 2]
        outs = refs[2 * npair + 2:2 * npair + 2 + npair]
        ia, ib = refs[2 * npair + 2 + npair:2 * npair + 4 + npair]
        rows_l = refs[2 * npair + 4 + npair:2 * npair + 4 + 2 * npair]
        sem = refs[-1]
        w = lax.axis_index("s") * _NC + lax.axis_index("c")

        @pl.loop(w, n_chunks, step=_NC * _NS)
        def _chunk(ch):
            pltpu.sync_copy(src_h.at[ch], ia)
            pltpu.sync_copy(dest_h.at[ch], ib)
            for t in range(npair):
                ta_h, tb_h, rows = tabs[2 * t], tabs[2 * t + 1], rows_l[t]
                cps = [pltpu.async_copy(ta_h.at[ia.at[j]],
                                        rows.at[pl.ds(j * _CW, _CW)], sem)
                       for j in range(_KR)]
                for cp in cps:
                    cp.wait()
                cps = [pltpu.async_copy(tb_h.at[ib.at[j]],
                                        rows.at[pl.ds(j * _CW, _CW)], sem,
                                        add=True)
                       for j in range(_KR)]
                for cp in cps:
                    cp.wait()
                pltpu.sync_copy(
                    rows, outs[t].at[pl.ds(ch * _KR * _CW, _KR * _CW)])

    flat = []
    for ta, tb in pairs:
        flat += [ta, tb]
    res = k(*flat, src2, dest2)
    return res if isinstance(res, (list, tuple)) else (res,)


def _sc_scatter(ea, dest2, zeros_h, n_nodes):
    mesh = plsc.VectorSubcoreMesh(core_axis_name="c", subcore_axis_name="s")
    kr = 4
    n_chunks = dest2.shape[0]
    rng = n_nodes // _NC
    sr = rng + 48
    zr = sr // _NS
    wb = (rng // _NS) // 8 * 8
    tail = rng - wb * _NS
    trash = rng

    @functools.partial(
        pl.kernel, mesh=mesh,
        out_type=jax.ShapeDtypeStruct((n_nodes, H), jnp.float32),
        compiler_params=pltpu.CompilerParams(use_tc_tiling_on_sc=False),
        scratch_types=[
            pltpu.VMEM_SHARED((sr, H), jnp.float32),
            pltpu.VMEM((kr, _CW), jnp.int32),
            pltpu.VMEM((kr, _CW), jnp.int32),
            pltpu.VMEM((kr * _CW, H), jnp.float32),
            pltpu.SemaphoreType.DMA,
        ],
    )
    def k(ea_h, dest_h, z_h, agg_h, acc, ib, i2, rows, sem):
        c = lax.axis_index("c")
        s = lax.axis_index("s")
        base = c * rng
        pltpu.sync_copy(z_h, acc.at[pl.ds(s * zr, zr)])
        plsc.subcore_barrier()

        @pl.loop(s, n_chunks, step=_NS)
        def _chunk(ch):
            pltpu.sync_copy(dest_h.at[ch], ib)
            pltpu.sync_copy(ea_h.at[pl.ds(ch * kr * _CW, kr * _CW)], rows)
            for j in range(kr):
                for q in range(_CW // 16):
                    dv = ib[j, pl.ds(q * 16, 16)]
                    rel = dv - base
                    ok = (rel >= 0) & (rel < rng)
                    i2[j, pl.ds(q * 16, 16)] = jnp.where(ok, rel, trash)
            for j in range(kr):
                pltpu.sync_copy(rows.at[pl.ds(j * _CW, _CW)],
                                acc.at[i2.at[j]], add=True)
        plsc.subcore_barrier()
        pltpu.sync_copy(acc.at[pl.ds(s * wb, wb)],
                        agg_h.at[pl.ds(base + s * wb, wb)])

        @pl.when(s == 0)
        def _tail():
            pltpu.sync_copy(acc.at[pl.ds(wb * _NS, tail)],
                            agg_h.at[pl.ds(base + wb * _NS, tail)])

    return k(ea, dest2, zeros_h)


def kernel(z, n, edge_index, q_0, params):
    N = z.shape[0]
    E = edge_index.shape[1]
    n_chunks = E // (_KR * _CW)
    src2 = edge_index[0].reshape(n_chunks, _KR, _CW)
    dest2 = edge_index[1].reshape(n_chunks, _KR, _CW)
    dest2s = edge_index[1].reshape(E // (4 * _CW), 4, _CW)
    rng = N // _NC
    zeros_h = jnp.zeros(((rng + 48) // _NS, H), jnp.float32)

    def pad16(a):
        return jnp.pad(a, ((0, 0), (0, 16 - a.shape[1])))

    z16, n16 = pad16(z), pad16(n)
    q32 = jnp.pad(q_0, ((0, 0), (0, H - 3)))
    nq32 = jnp.pad(-q_0, ((0, 0), (0, H - 3)))

    p = params
    (wz6, b1), (w2, b2), (w3, b3) = p['enc_node']
    (we1, be1), (we2, be2), (we3, be3) = p['enc_edge']
    wz16 = jnp.zeros((16, H), jnp.float32).at[:3].set(wz6[:3])
    wn16 = jnp.zeros((16, H), jnp.float32).at[:3].set(wz6[3:])
    we1p = jnp.zeros((H, H), jnp.float32).at[:3].set(we1[:3])

    def row(b):
        return b.reshape(1, -1)

    def bd4(w):
        return jax.scipy.linalg.block_diag(w, w, w, w)

    def row4(b):
        return jnp.tile(b.reshape(1, -1), (1, 4))

    msk4 = bd4(jnp.ones((H, H), jnp.float32))

    e0 = p['edge_0']
    e1 = p['edge_1']
    n0 = p['node_0']
    n1 = p['node_1']
    dec = p['dec']

    Bn = 4000
    Be = 5000

    x0, s0, d0 = _tc_call(
        _node_enc_body, N, Bn, [z16, n16],
        [wz16, wn16, row(b1), w2, row(b2), w3, row(b3),
         e0[0][0][H:2 * H], e0[0][0][2 * H:]],
        [H, H, H])

    g0, u0 = _sc_gather([(s0, d0), (q32, nq32)], src2, dest2, n_chunks)
    g0p = g0.reshape(E // 4, 4 * H)
    u0p = u0.reshape(E // 4, 4 * H)
    ea0p, eat1p = _tc_call(
        _edge0_body, E // 4, Be, [g0p, u0p],
        [msk4, bd4(we1p), row4(we1[3]), row4(be1), bd4(we2), row4(be2),
         bd4(we3), row4(be3),
         bd4(e0[0][0][:H]), row4(e0[0][1]), bd4(e0[1][0]), row4(e0[1][1]),
         bd4(e0[2][0]), row4(e0[2][1])],
        [4 * H, 4 * H])
    agg0 = _sc_scatter(ea0p.reshape(E, H), dest2s, zeros_h, N)
    x1, s1, d1 = _tc_call(
        _node_upd_body, N, Bn, [x0, agg0],
        [n0[0][0][:H], n0[0][0][H:], row(n0[0][1]), n0[1][0], row(n0[1][1]),
         n0[2][0], row(n0[2][1]), e1[0][0][H:2 * H], e1[0][0][2 * H:]],
        [H, H, H])

    (g1,) = _sc_gather([(s1, d1)], src2, dest2, n_chunks)
    g1p = g1.reshape(E // 4, 4 * H)
    (ea1p,) = _tc_call(
        _edge1_body, E // 4, Be, [eat1p, g1p],
        [bd4(e1[0][0][:H]), row4(e1[0][1]), bd4(e1[1][0]), row4(e1[1][1]),
         bd4(e1[2][0]), row4(e1[2][1])],
        [4 * H])
    agg1 = _sc_scatter(ea1p.reshape(E, H), dest2s, zeros_h, N)
    (out,) = _tc_call(
        _node_dec_body, N, Bn, [x1, agg1],
        [n1[0][0][:H], n1[0][0][H:], row(n1[0][1]), n1[1][0], row(n1[1][1]),
         n1[2][0], row(n1[2][1]),
         dec[0][0], row(dec[0][1]), dec[1][0], row(dec[1][1]),
         dec[2][0], row(dec[2][1])],
        [3])
    return out


# speedup vs baseline: 7.6442x; 1.0439x over previous
import functools

import jax
import jax.numpy as jnp
from jax import lax
from jax.experimental import pallas as pl
from jax.experimental.pallas import tpu as pltpu
from jax.experimental.pallas import tpu_sc as plsc

H = 32
_SILU = jax.nn.silu

_NC = 2
_NS = 16
_CW = 128
_KR = 10


def _node_enc_body(z, n, wz, wn, b1, w2, b2, w3, b3, ws, wd,
                   x0_o, s_o, d_o):
    h = _SILU(z[...] @ wz[...] + n[...] @ wn[...] + b1[...])
    h = _SILU(h @ w2[...] + b2[...])
    x0 = h @ w3[...] + b3[...]
    x0_o[...] = x0
    s_o[...] = x0 @ ws[...]
    d_o[...] = x0 @ wd[...]


def _edge0_body(g, up, msk, we1p, we1n, be1, we2, be2, we3, be3,
                w1e, b1, w2, b2, w3, b3, ea_o, eat_o):
    u = up[...]
    norm = jnp.sqrt((u * u) @ msk[...])
    e = _SILU(u @ we1p[...] + norm * we1n[...] + be1[...])
    e = _SILU(e @ we2[...] + be2[...])
    e = e @ we3[...] + be3[...]
    h = _SILU(e @ w1e[...] + g[...] + b1[...])
    h = _SILU(h @ w2[...] + b2[...])
    ea = h @ w3[...] + b3[...]
    ea_o[...] = ea
    eat_o[...] = e + ea


def _edge1_body(eat, g, w1e, b1, w2, b2, w3, b3, ea_o):
    h = _SILU(eat[...] @ w1e[...] + g[...] + b1[...])
    h = _SILU(h @ w2[...] + b2[...])
    ea_o[...] = h @ w3[...] + b3[...]


def _node_upd_body(x, a, w1x, w1a, b1, w2, b2, w3, b3, ws, wd,
                   x1_o, s_o, d_o):
    h = _SILU(x[...] @ w1x[...] + a[...] @ w1a[...] + b1[...])
    h = _SILU(h @ w2[...] + b2[...])
    x1 = x[...] + (h @ w3[...] + b3[...])
    x1_o[...] = x1
    s_o[...] = x1 @ ws[...]
    d_o[...] = x1 @ wd[...]


def _node_dec_body(x, a, w1x, w1a, b1, w2, b2, w3, b3,
                   wd1, bd1, wd2, bd2, wd3, bd3, o):
    h = _SILU(x[...] @ w1x[...] + a[...] @ w1a[...] + b1[...])
    h = _SILU(h @ w2[...] + b2[...])
    x2 = x[...] + (h @ w3[...] + b3[...])
    h = _SILU(x2 @ wd1[...] + bd1[...])
    h = _SILU(h @ wd2[...] + bd2[...])
    o[...] = h @ wd3[...] + bd3[...]


def _tc_call(body, n_rows, blk, data, weights, out_widths):
    specs = [pl.BlockSpec((blk, d.shape[1]), lambda i: (i, 0)) for d in data]
    specs += [pl.BlockSpec(w.shape, lambda i, _r=w.ndim: (0,) * _r)
              for w in weights]
    return pl.pallas_call(
        body,
        grid=(n_rows // blk,),
        in_specs=specs,
        out_specs=[pl.BlockSpec((blk, wd), lambda i: (i, 0))
                   for wd in out_widths],
        out_shape=[jax.ShapeDtypeStruct((n_rows, wd), jnp.float32)
                   for wd in out_widths],
    )(*data, *weights)


def _sc_gather(pairs, src2, dest2, kr):
    mesh = plsc.VectorSubcoreMesh(core_axis_name="c", subcore_axis_name="s")
    n_chunks = src2.shape[0]
    cs = kr * _CW
    ecnt = n_chunks * cs
    dims = [ta.shape[1] for ta, _ in pairs]
    npair = len(dims)
    step = _NC * _NS

    @functools.partial(
        pl.kernel, mesh=mesh,
        out_type=[jax.ShapeDtypeStruct((ecnt, d), jnp.float32) for d in dims],
        compiler_params=pltpu.CompilerParams(use_tc_tiling_on_sc=False),
        scratch_types=[
            [[pltpu.VMEM((kr, _CW), jnp.int32) for _ in range(2)]
             for _ in range(2)],
            [[pltpu.VMEM((cs, d), jnp.float32) for _ in range(2)]
             for d in dims],
            pltpu.SemaphoreType.DMA,
            pltpu.SemaphoreType.DMA,
            pltpu.SemaphoreType.DMA,
            [pltpu.SemaphoreType.DMA for _ in range(2)],
        ],
    )
    def k(*refs):
        tabs = refs[:2 * npair]
        src_h, dest_h = refs[2 * npair:2 * npair + 2]
        outs = refs[2 * npair + 2:2 * npair + 2 + npair]
        idxb, rowsb, semi, sema, semb, semo = refs[2 * npair + 2 + npair:]
        w = lax.axis_index("s") * _NC + lax.axis_index("c")

        pltpu.sync_copy(src_h.at[w], idxb[0][0])
        pltpu.sync_copy(dest_h.at[w], idxb[0][1])

        def do_chunk(c, p):
            ia, ib = idxb[p]
            @pl.when(c + step < n_chunks)
            def _pf():
                pltpu.async_copy(src_h.at[c + step], idxb[1 - p][0], semi)
                pltpu.async_copy(dest_h.at[c + step], idxb[1 - p][1], semi)
            @pl.when(c >= w + 2 * step)
            def _drain_store():
                for t in range(npair):
                    pltpu.make_async_copy(
                        rowsb[t][p], outs[t].at[pl.ds(0, cs)], semo[p]).wait()
            for t in range(npair):
                for j in range(kr):
                    pltpu.async_copy(tabs[2 * t].at[ia.at[j]],
                                     rowsb[t][p].at[pl.ds(j * _CW, _CW)],
                                     sema)
            for t in range(npair):
                pltpu.make_async_copy(
                    tabs[0].at[ia.at[0]], rowsb[t][p], sema).wait()
            for t in range(npair):
                for j in range(kr):
                    pltpu.async_copy(tabs[2 * t + 1].at[ib.at[j]],
                                     rowsb[t][p].at[pl.ds(j * _CW, _CW)],
                                     semb, add=True)
            for t in range(npair):
                pltpu.make_async_copy(
                    tabs[1].at[ib.at[0]], rowsb[t][p], semb).wait()
            for t in range(npair):
                pltpu.async_copy(rowsb[t][p],
                                 outs[t].at[pl.ds(c * cs, cs)], semo[p])
            @pl.when(c + step < n_chunks)
            def _wi():
                pltpu.make_async_copy(src_h.at[c], idxb[1 - p][0], semi).wait()
                pltpu.make_async_copy(dest_h.at[c], idxb[1 - p][1],
                                      semi).wait()

        @pl.loop(w, n_chunks, step=2 * step)
        def _chunk2(c):
            do_chunk(c, 0)

            @pl.when(c + step < n_chunks)
            def _second():
                do_chunk(c + step, 1)

        for p in (0, 1):
            for t in range(npair):
                pltpu.make_async_copy(
                    rowsb[t][p], outs[t].at[pl.ds(0, cs)], semo[p]).wait()

    flat = []
    for ta, tb in pairs:
        flat += [ta, tb]
    res = k(*flat, src2, dest2)
    return res if isinstance(res, (list, tuple)) else (res,)


def _sc_scatter(ea, dest2, zeros_h, n_nodes):
    mesh = plsc.VectorSubcoreMesh(core_axis_name="c", subcore_axis_name="s")
    kr = 4
    cs = kr * _CW
    n_chunks = dest2.shape[0]
    rng = n_nodes // _NC
    sr = rng + 48
    zr = sr // _NS
    wb = (rng // _NS) // 8 * 8
    tail = rng - wb * _NS
    trash = rng
    step = _NS

    @functools.partial(
        pl.kernel, mesh=mesh,
        out_type=jax.ShapeDtypeStruct((n_nodes, H), jnp.float32),
        compiler_params=pltpu.CompilerParams(use_tc_tiling_on_sc=False),
        scratch_types=[
            pltpu.VMEM_SHARED((sr, H), jnp.float32),
            [pltpu.VMEM((kr, _CW), jnp.int32) for _ in range(2)],
            [pltpu.VMEM((kr, _CW), jnp.int32) for _ in range(2)],
            pltpu.VMEM((cs, H), jnp.float32),
            pltpu.SemaphoreType.DMA,
            pltpu.SemaphoreType.DMA,
            pltpu.SemaphoreType.DMA,
        ],
    )
    def k(ea_h, dest_h, z_h, agg_h, acc, ibs, i2s, rows, semi, semr, sems):
        c = lax.axis_index("c")
        s = lax.axis_index("s")
        base = c * rng
        pltpu.sync_copy(z_h, acc.at[pl.ds(s * zr, zr)])
        pltpu.sync_copy(dest_h.at[s], ibs[0])
        plsc.subcore_barrier()

        def do_chunk(ch, p):
            ib, i2 = ibs[p], i2s[p]
            @pl.when(ch + step < n_chunks)
            def _pf():
                pltpu.async_copy(dest_h.at[ch + step], ibs[1 - p], semi)
            @pl.when(ch >= s + step)
            def _drain_s():
                for j in range(kr):
                    pltpu.make_async_copy(
                        rows.at[pl.ds(j * _CW, _CW)],
                        acc.at[i2s[1 - p].at[j]], sems).wait()
            pltpu.async_copy(ea_h.at[pl.ds(ch * cs, cs)], rows, semr)
            for j in range(kr):
                for q in range(_CW // 16):
                    dv = ib[j, pl.ds(q * 16, 16)]
                    rel = dv - base
                    ok = (rel >= 0) & (rel < rng)
                    i2[j, pl.ds(q * 16, 16)] = jnp.where(ok, rel, trash)
            pltpu.make_async_copy(ea_h.at[pl.ds(0, cs)], rows, semr).wait()
            for j in range(kr):
                pltpu.async_copy(rows.at[pl.ds(j * _CW, _CW)],
                                 acc.at[i2.at[j]], sems, add=True)
            @pl.when(ch + step < n_chunks)
            def _wi():
                pltpu.make_async_copy(dest_h.at[ch], ibs[1 - p], semi).wait()

        @pl.loop(s, n_chunks, step=2 * step)
        def _chunk2(ch):
            do_chunk(ch, 0)

            @pl.when(ch + step < n_chunks)
            def _second():
                do_chunk(ch + step, 1)

        for j in range(kr):
            pltpu.make_async_copy(rows.at[pl.ds(j * _CW, _CW)],
                                  acc.at[i2s[1].at[j]], sems).wait()
        plsc.subcore_barrier()
        pltpu.sync_copy(acc.at[pl.ds(s * wb, wb)],
                        agg_h.at[pl.ds(base + s * wb, wb)])

        @pl.when(s == 0)
        def _tail():
            pltpu.sync_copy(acc.at[pl.ds(wb * _NS, tail)],
                            agg_h.at[pl.ds(base + wb * _NS, tail)])

    return k(ea, dest2, zeros_h)


def kernel(z, n, edge_index, q_0, params):
    N = z.shape[0]
    E = edge_index.shape[1]
    src2a = edge_index[0].reshape(E // (5 * _CW), 5, _CW)
    dest2a = edge_index[1].reshape(E // (5 * _CW), 5, _CW)
    src2b = edge_index[0].reshape(E // (10 * _CW), 10, _CW)
    dest2b = edge_index[1].reshape(E // (10 * _CW), 10, _CW)
    dest2s = edge_index[1].reshape(E // (4 * _CW), 4, _CW)
    rng = N // _NC
    zeros_h = jnp.zeros(((rng + 48) // _NS, H), jnp.float32)

    def pad16(a):
        return jnp.pad(a, ((0, 0), (0, 16 - a.shape[1])))

    z16, n16 = pad16(z), pad16(n)
    q32 = jnp.pad(q_0, ((0, 0), (0, H - 3)))
    nq32 = jnp.pad(-q_0, ((0, 0), (0, H - 3)))

    p = params
    (wz6, b1), (w2, b2), (w3, b3) = p['enc_node']
    (we1, be1), (we2, be2), (we3, be3) = p['enc_edge']
    wz16 = jnp.zeros((16, H), jnp.float32).at[:3].set(wz6[:3])
    wn16 = jnp.zeros((16, H), jnp.float32).at[:3].set(wz6[3:])
    we1p = jnp.zeros((H, H), jnp.float32).at[:3].set(we1[:3])

    def row(b):
        return b.reshape(1, -1)

    def bd4(w):
        return jax.scipy.linalg.block_diag(w, w, w, w)

    def row4(b):
        return jnp.tile(b.reshape(1, -1), (1, 4))

    msk4 = bd4(jnp.ones((H, H), jnp.float32))

    e0 = p['edge_0']
    e1 = p['edge_1']
    n0 = p['node_0']
    n1 = p['node_1']
    dec = p['dec']

    Bn = 4000
    Be = 5000

    x0, s0, d0 = _tc_call(
        _node_enc_body, N, Bn, [z16, n16],
        [wz16, wn16, row(b1), w2, row(b2), w3, row(b3),
         e0[0][0][H:2 * H], e0[0][0][2 * H:]],
        [H, H, H])

    g0, u0 = _sc_gather([(s0, d0), (q32, nq32)], src2a, dest2a, 5)
    g0p = g0.reshape(E // 4, 4 * H)
    u0p = u0.reshape(E // 4, 4 * H)
    ea0p, eat1p = _tc_call(
        _edge0_body, E // 4, Be, [g0p, u0p],
        [msk4, bd4(we1p), row4(we1[3]), row4(be1), bd4(we2), row4(be2),
         bd4(we3), row4(be3),
         bd4(e0[0][0][:H]), row4(e0[0][1]), bd4(e0[1][0]), row4(e0[1][1]),
         bd4(e0[2][0]), row4(e0[2][1])],
        [4 * H, 4 * H])
    agg0 = _sc_scatter(ea0p.reshape(E, H), dest2s, zeros_h, N)
    x1, s1, d1 = _tc_call(
        _node_upd_body, N, Bn, [x0, agg0],
        [n0[0][0][:H], n0[0][0][H:], row(n0[0][1]), n0[1][0], row(n0[1][1]),
         n0[2][0], row(n0[2][1]), e1[0][0][H:2 * H], e1[0][0][2 * H:]],
        [H, H, H])

    (g1,) = _sc_gather([(s1, d1)], src2b, dest2b, 10)
    g1p = g1.reshape(E // 4, 4 * H)
    (ea1p,) = _tc_call(
        _edge1_body, E // 4, Be, [eat1p, g1p],
        [bd4(e1[0][0][:H]), row4(e1[0][1]), bd4(e1[1][0]), row4(e1[1][1]),
         bd4(e1[2][0]), row4(e1[2][1])],
        [4 * H])
    agg1 = _sc_scatter(ea1p.reshape(E, H), dest2s, zeros_h, N)
    (out,) = _tc_call(
        _node_dec_body, N, Bn, [x1, agg1],
        [n1[0][0][:H], n1[0][0][H:], row(n1[0][1]), n1[1][0], row(n1[1][1]),
         n1[2][0], row(n1[2][1]),
         dec[0][0], row(dec[0][1]), dec[1][0], row(dec[1][1]),
         dec[2][0], row(dec[2][1])],
        [3])
    return out


# speedup vs baseline: 10.5054x; 1.3743x over previous
import functools

import jax
import jax.numpy as jnp
from jax import lax
from jax.experimental import pallas as pl
from jax.experimental.pallas import tpu as pltpu
from jax.experimental.pallas import tpu_sc as plsc

H = 32
_SILU = jax.nn.silu

_NC = 2
_NS = 16
_CW = 128
_KR = 10


def _node_enc_body(z, n, wz, wn, b1, w2, b2, w3, b3, ws, wd,
                   x0_o, s_o, d_o):
    h = _SILU(z[...] @ wz[...] + n[...] @ wn[...] + b1[...])
    h = _SILU(h @ w2[...] + b2[...])
    x0 = h @ w3[...] + b3[...]
    x0_o[...] = x0
    s_o[...] = x0 @ ws[...]
    d_o[...] = x0 @ wd[...]


def _edge0_body(g, up, msk, we1p, we1n, be1, we2, be2, we3, be3,
                w1e, b1, w2, b2, w3, b3, ea_o, eat_o):
    u = up[...]
    norm = jnp.sqrt((u * u) @ msk[...])
    e = _SILU(u @ we1p[...] + norm * we1n[...] + be1[...])
    e = _SILU(e @ we2[...] + be2[...])
    e = e @ we3[...] + be3[...]
    h = _SILU(e @ w1e[...] + g[...] + b1[...])
    h = _SILU(h @ w2[...] + b2[...])
    ea = h @ w3[...] + b3[...]
    ea_o[...] = ea
    eat_o[...] = e + ea


def _edge1_body(eat, g, w1e, b1, w2, b2, w3, b3, ea_o):
    h = _SILU(eat[...] @ w1e[...] + g[...] + b1[...])
    h = _SILU(h @ w2[...] + b2[...])
    ea_o[...] = h @ w3[...] + b3[...]


def _node_upd_body(x, a, w1x, w1a, b1, w2, b2, w3, b3, ws, wd,
                   x1_o, s_o, d_o):
    h = _SILU(x[...] @ w1x[...] + a[...] @ w1a[...] + b1[...])
    h = _SILU(h @ w2[...] + b2[...])
    x1 = x[...] + (h @ w3[...] + b3[...])
    x1_o[...] = x1
    s_o[...] = x1 @ ws[...]
    d_o[...] = x1 @ wd[...]


def _node_dec_body(x, a, w1x, w1a, b1, w2, b2, w3, b3,
                   wd1, bd1, wd2, bd2, wd3, bd3, o):
    h = _SILU(x[...] @ w1x[...] + a[...] @ w1a[...] + b1[...])
    h = _SILU(h @ w2[...] + b2[...])
    x2 = x[...] + (h @ w3[...] + b3[...])
    h = _SILU(x2 @ wd1[...] + bd1[...])
    h = _SILU(h @ wd2[...] + bd2[...])
    o[...] = h @ wd3[...] + bd3[...]


def _tc_call(body, n_rows, blk, data, weights, out_widths):
    specs = [pl.BlockSpec((blk, d.shape[1]), lambda i: (i, 0)) for d in data]
    specs += [pl.BlockSpec(w.shape, lambda i, _r=w.ndim: (0,) * _r)
              for w in weights]
    return pl.pallas_call(
        body,
        grid=(n_rows // blk,),
        in_specs=specs,
        out_specs=[pl.BlockSpec((blk, wd), lambda i: (i, 0))
                   for wd in out_widths],
        out_shape=[jax.ShapeDtypeStruct((n_rows, wd), jnp.float32)
                   for wd in out_widths],
    )(*data, *weights)


def _sc_gather(pairs, src2, dest2, kr):
    mesh = plsc.VectorSubcoreMesh(core_axis_name="c", subcore_axis_name="s")
    n_chunks = src2.shape[0]
    cs = kr * _CW
    ecnt = n_chunks * cs
    dims = [ta.shape[1] for ta, _ in pairs]
    npair = len(dims)
    step = _NC * _NS

    @functools.partial(
        pl.kernel, mesh=mesh,
        out_type=[jax.ShapeDtypeStruct((ecnt, d), jnp.float32) for d in dims],
        compiler_params=pltpu.CompilerParams(use_tc_tiling_on_sc=False),
        scratch_types=[
            [[pltpu.VMEM((kr, _CW), jnp.int32) for _ in range(2)]
             for _ in range(2)],
            [[pltpu.VMEM((cs, d), jnp.float32) for _ in range(2)]
             for d in dims],
            pltpu.SemaphoreType.DMA,
            pltpu.SemaphoreType.DMA,
            pltpu.SemaphoreType.DMA,
            [pltpu.SemaphoreType.DMA for _ in range(2)],
        ],
    )
    def k(*refs):
        tabs = refs[:2 * npair]
        src_h, dest_h = refs[2 * npair:2 * npair + 2]
        outs = refs[2 * npair + 2:2 * npair + 2 + npair]
        idxb, rowsb, semi, sema, semb, semo = refs[2 * npair + 2 + npair:]
        w = lax.axis_index("s") * _NC + lax.axis_index("c")

        pltpu.sync_copy(src_h.at[w], idxb[0][0])
        pltpu.sync_copy(dest_h.at[w], idxb[0][1])

        def do_chunk(c, p):
            ia, ib = idxb[p]
            @pl.when(c + step < n_chunks)
            def _pf():
                pltpu.async_copy(src_h.at[c + step], idxb[1 - p][0], semi)
                pltpu.async_copy(dest_h.at[c + step], idxb[1 - p][1], semi)
            @pl.when(c >= w + 2 * step)
            def _drain_store():
                for t in range(npair):
                    pltpu.make_async_copy(
                        rowsb[t][p], outs[t].at[pl.ds(0, cs)], semo[p]).wait()
            for t in range(npair):
                for j in range(kr):
                    pltpu.async_copy(tabs[2 * t].at[ia.at[j]],
                                     rowsb[t][p].at[pl.ds(j * _CW, _CW)],
                                     sema)
            for t in range(npair):
                pltpu.make_async_copy(
                    tabs[0].at[ia.at[0]], rowsb[t][p], sema).wait()
            for t in range(npair):
                for j in range(kr):
                    pltpu.async_copy(tabs[2 * t + 1].at[ib.at[j]],
                                     rowsb[t][p].at[pl.ds(j * _CW, _CW)],
                                     semb, add=True)
            for t in range(npair):
                pltpu.make_async_copy(
                    tabs[1].at[ib.at[0]], rowsb[t][p], semb).wait()
            for t in range(npair):
                pltpu.async_copy(rowsb[t][p],
                                 outs[t].at[pl.ds(c * cs, cs)], semo[p])
            @pl.when(c + step < n_chunks)
            def _wi():
                pltpu.make_async_copy(src_h.at[c], idxb[1 - p][0], semi).wait()
                pltpu.make_async_copy(dest_h.at[c], idxb[1 - p][1],
                                      semi).wait()

        @pl.loop(w, n_chunks, step=2 * step)
        def _chunk2(c):
            do_chunk(c, 0)

            @pl.when(c + step < n_chunks)
            def _second():
                do_chunk(c + step, 1)

        for p in (0, 1):
            for t in range(npair):
                pltpu.make_async_copy(
                    rowsb[t][p], outs[t].at[pl.ds(0, cs)], semo[p]).wait()

    flat = []
    for ta, tb in pairs:
        flat += [ta, tb]
    res = k(*flat, src2, dest2)
    return res if isinstance(res, (list, tuple)) else (res,)


def _sc_scatter(ea, dest2, zeros_h, n_nodes):
    mesh = plsc.VectorSubcoreMesh(core_axis_name="c", subcore_axis_name="s")
    kr = 4
    cs = kr * _CW
    n_chunks = dest2.shape[0]
    rng = n_nodes // _NC
    sr = rng + 48
    zr = sr // _NS
    wb = (rng // _NS) // 8 * 8
    tail = rng - wb * _NS
    trash = rng
    step = _NS

    @functools.partial(
        pl.kernel, mesh=mesh,
        out_type=jax.ShapeDtypeStruct((n_nodes, H), jnp.float32),
        compiler_params=pltpu.CompilerParams(use_tc_tiling_on_sc=False),
        scratch_types=[
            pltpu.VMEM_SHARED((sr, H), jnp.float32),
            [pltpu.VMEM((kr, _CW), jnp.int32) for _ in range(2)],
            [pltpu.VMEM((kr, _CW), jnp.int32) for _ in range(2)],
            pltpu.VMEM((cs, H), jnp.float32),
            pltpu.SemaphoreType.DMA,
            pltpu.SemaphoreType.DMA,
            pltpu.SemaphoreType.DMA,
        ],
    )
    def k(ea_h, dest_h, z_h, agg_h, acc, ibs, i2s, rows, semi, semr, sems):
        c = lax.axis_index("c")
        s = lax.axis_index("s")
        base = c * rng
        pltpu.sync_copy(z_h, acc.at[pl.ds(s * zr, zr)])
        pltpu.sync_copy(dest_h.at[s], ibs[0])
        plsc.subcore_barrier()

        def do_chunk(ch, p):
            ib, i2 = ibs[p], i2s[p]
            @pl.when(ch + step < n_chunks)
            def _pf():
                pltpu.async_copy(dest_h.at[ch + step], ibs[1 - p], semi)
            @pl.when(ch >= s + step)
            def _drain_s():
                for j in range(kr):
                    pltpu.make_async_copy(
                        rows.at[pl.ds(j * _CW, _CW)],
                        acc.at[i2s[1 - p].at[j]], sems).wait()
            pltpu.async_copy(ea_h.at[pl.ds(ch * cs, cs)], rows, semr)
            iot = lax.iota(jnp.int32, 16)
            for j in range(kr):
                for q in range(_CW // 16):
                    dv = ib[j, pl.ds(q * 16, 16)]
                    rel = dv - base
                    ok = (rel >= 0) & (rel < rng)
                    i2[j, pl.ds(q * 16, 16)] = jnp.where(
                        ok, rel, trash + (q % 3) * 16 + iot)
            pltpu.make_async_copy(ea_h.at[pl.ds(0, cs)], rows, semr).wait()
            for j in range(kr):
                pltpu.async_copy(rows.at[pl.ds(j * _CW, _CW)],
                                 acc.at[i2.at[j]], sems, add=True)
            @pl.when(ch + step < n_chunks)
            def _wi():
                pltpu.make_async_copy(dest_h.at[ch], ibs[1 - p], semi).wait()

        @pl.loop(s, n_chunks, step=2 * step)
        def _chunk2(ch):
            do_chunk(ch, 0)

            @pl.when(ch + step < n_chunks)
            def _second():
                do_chunk(ch + step, 1)

        for j in range(kr):
            pltpu.make_async_copy(rows.at[pl.ds(j * _CW, _CW)],
                                  acc.at[i2s[1].at[j]], sems).wait()
        plsc.subcore_barrier()
        pltpu.sync_copy(acc.at[pl.ds(s * wb, wb)],
                        agg_h.at[pl.ds(base + s * wb, wb)])

        @pl.when(s == 0)
        def _tail():
            pltpu.sync_copy(acc.at[pl.ds(wb * _NS, tail)],
                            agg_h.at[pl.ds(base + wb * _NS, tail)])

    return k(ea, dest2, zeros_h)


def kernel(z, n, edge_index, q_0, params):
    N = z.shape[0]
    E = edge_index.shape[1]
    src2a = edge_index[0].reshape(E // (5 * _CW), 5, _CW)
    dest2a = edge_index[1].reshape(E // (5 * _CW), 5, _CW)
    src2b = edge_index[0].reshape(E // (10 * _CW), 10, _CW)
    dest2b = edge_index[1].reshape(E // (10 * _CW), 10, _CW)
    dest2s = edge_index[1].reshape(E // (4 * _CW), 4, _CW)
    rng = N // _NC
    zeros_h = jnp.zeros(((rng + 48) // _NS, H), jnp.float32)

    def pad16(a):
        return jnp.pad(a, ((0, 0), (0, 16 - a.shape[1])))

    z16, n16 = pad16(z), pad16(n)
    q32 = jnp.pad(q_0, ((0, 0), (0, H - 3)))
    nq32 = jnp.pad(-q_0, ((0, 0), (0, H - 3)))

    p = params
    (wz6, b1), (w2, b2), (w3, b3) = p['enc_node']
    (we1, be1), (we2, be2), (we3, be3) = p['enc_edge']
    wz16 = jnp.zeros((16, H), jnp.float32).at[:3].set(wz6[:3])
    wn16 = jnp.zeros((16, H), jnp.float32).at[:3].set(wz6[3:])
    we1p = jnp.zeros((H, H), jnp.float32).at[:3].set(we1[:3])

    def row(b):
        return b.reshape(1, -1)

    def bd4(w):
        return jax.scipy.linalg.block_diag(w, w, w, w)

    def row4(b):
        return jnp.tile(b.reshape(1, -1), (1, 4))

    msk4 = bd4(jnp.ones((H, H), jnp.float32))

    e0 = p['edge_0']
    e1 = p['edge_1']
    n0 = p['node_0']
    n1 = p['node_1']
    dec = p['dec']

    Bn = 4000
    Be = 5000

    x0, s0, d0 = _tc_call(
        _node_enc_body, N, Bn, [z16, n16],
        [wz16, wn16, row(b1), w2, row(b2), w3, row(b3),
         e0[0][0][H:2 * H], e0[0][0][2 * H:]],
        [H, H, H])

    g0, u0 = _sc_gather([(s0, d0), (q32, nq32)], src2a, dest2a, 5)
    g0p = g0.reshape(E // 4, 4 * H)
    u0p = u0.reshape(E // 4, 4 * H)
    ea0p, eat1p = _tc_call(
        _edge0_body, E // 4, Be, [g0p, u0p],
        [msk4, bd4(we1p), row4(we1[3]), row4(be1), bd4(we2), row4(be2),
         bd4(we3), row4(be3),
         bd4(e0[0][0][:H]), row4(e0[0][1]), bd4(e0[1][0]), row4(e0[1][1]),
         bd4(e0[2][0]), row4(e0[2][1])],
        [4 * H, 4 * H])
    agg0 = _sc_scatter(ea0p.reshape(E, H), dest2s, zeros_h, N)
    x1, s1, d1 = _tc_call(
        _node_upd_body, N, Bn, [x0, agg0],
        [n0[0][0][:H], n0[0][0][H:], row(n0[0][1]), n0[1][0], row(n0[1][1]),
         n0[2][0], row(n0[2][1]), e1[0][0][H:2 * H], e1[0][0][2 * H:]],
        [H, H, H])

    (g1,) = _sc_gather([(s1, d1)], src2b, dest2b, 10)
    g1p = g1.reshape(E // 4, 4 * H)
    (ea1p,) = _tc_call(
        _edge1_body, E // 4, Be, [eat1p, g1p],
        [bd4(e1[0][0][:H]), row4(e1[0][1]), bd4(e1[1][0]), row4(e1[1][1]),
         bd4(e1[2][0]), row4(e1[2][1])],
        [4 * H])
    agg1 = _sc_scatter(ea1p.reshape(E, H), dest2s, zeros_h, N)
    (out,) = _tc_call(
        _node_dec_body, N, Bn, [x1, agg1],
        [n1[0][0][:H], n1[0][0][H:], row(n1[0][1]), n1[1][0], row(n1[1][1]),
         n1[2][0], row(n1[2][1]),
         dec[0][0], row(dec[0][1]), dec[1][0], row(dec[1][1]),
         dec[2][0], row(dec[2][1])],
        [3])
    return out
